# baseline (speedup 1.0000x reference)
import functools

import jax
import jax.numpy as jnp
from jax import lax
from jax.experimental import pallas as pl
from jax.experimental.pallas import tpu as pltpu
from jax.experimental.pallas import tpu_sc as plsc

_LANES = 128


_UNROLL = 28


def _nn_body(nstep, qb, y_ref, gx_ref, gy_ref, sig_ref, idx_ref, g_ref):
    yx = y_ref[:, 0:1]
    yy = y_ref[:, 1:2]
    lane = lax.broadcasted_iota(jnp.int32, (1, _LANES), 1).astype(jnp.float32)

    def one(c, bv, bi):
        off = pl.multiple_of(c * _LANES, _LANES)
        gx = gx_ref[:, pl.ds(off, _LANES)]
        gy = gy_ref[:, pl.ds(off, _LANES)]
        dx = yx - gx
        dy = yy - gy
        d = dx * dx + dy * dy
        upd = d < bv
        ki = jnp.float32(_LANES) * lax.convert_element_type(c, jnp.float32) + lane
        return jnp.where(upd, d, bv), jnp.where(upd, ki, bi)

    def step(s, carry):
        bv, bi = carry
        for u in range(_UNROLL):
            bv, bi = one(s * _UNROLL + u, bv, bi)
        return bv, bi

    bv0 = jnp.full((qb, _LANES), jnp.inf, jnp.float32)
    bi0 = jnp.zeros((qb, _LANES), jnp.float32)
    bv, bi = lax.fori_loop(0, nstep // _UNROLL, step, (bv0, bi0))

    m = jnp.min(bv, axis=1, keepdims=True)
    big = jnp.float32(3.0e38)
    idxf = jnp.min(jnp.where(bv == m, bi, big), axis=1, keepdims=True)
    idx_ref[...] = idxf.astype(jnp.int32)
    g_ref[...] = jnp.broadcast_to(jnp.exp(sig_ref[0, 0]), g_ref.shape)


def _nn_argmin(y, gxp, gyp, sig):
    q = y.shape[0]
    kpad = gxp.shape[1]
    qb = 2048
    nstep = kpad // _LANES
    body = functools.partial(_nn_body, nstep, qb)
    return pl.pallas_call(
        body,
        grid=(q // qb,),
        in_specs=[
            pl.BlockSpec((qb, 2), lambda i: (i, 0)),
            pl.BlockSpec((1, kpad), lambda i: (0, 0)),
            pl.BlockSpec((1, kpad), lambda i: (0, 0)),
            pl.BlockSpec((1, 1), lambda i: (0, 0)),
        ],
        out_specs=[
            pl.BlockSpec((qb, 1), lambda i: (i, 0)),
            pl.BlockSpec((qb, 2), lambda i: (i, 0)),
        ],
        out_shape=[
            jax.ShapeDtypeStruct((q, 1), jnp.int32),
            jax.ShapeDtypeStruct((q, 2), jnp.float32),
        ],
    )(y, gxp, gyp, sig)


def _sc_gather(idx, u_grid, v_grid):
    q = idx.shape[0]
    info = plsc.get_sparse_core_info()
    nw = info.num_cores * info.num_subcores
    bpw = q // nw
    mesh = plsc.VectorSubcoreMesh(core_axis_name="c", subcore_axis_name="s")

    @functools.partial(
        pl.kernel,
        mesh=mesh,
        out_type=[
            jax.ShapeDtypeStruct((q,), jnp.float32),
            jax.ShapeDtypeStruct((q,), jnp.float32),
        ],
        scratch_types=[
            pltpu.VMEM((bpw,), jnp.int32),
            pltpu.VMEM((bpw,), jnp.float32),
            pltpu.VMEM((bpw,), jnp.float32),
            pltpu.SemaphoreType.DMA,
            pltpu.SemaphoreType.DMA,
        ],
    )
    def gather_k(idx_hbm, u_hbm, v_hbm, uo_hbm, vo_hbm, idx_v, u_v, v_v, s1, s2):
        wid = lax.axis_index("s") * info.num_cores + lax.axis_index("c")
        base = wid * bpw
        pltpu.sync_copy(idx_hbm.at[pl.ds(base, bpw)], idx_v)
        cu = pltpu.async_copy(u_hbm.at[idx_v], u_v, s1)
        cv = pltpu.async_copy(v_hbm.at[idx_v], v_v, s2)
        cu.wait()
        cv.wait()
        pltpu.sync_copy(u_v, uo_hbm.at[pl.ds(base, bpw)])
        pltpu.sync_copy(v_v, vo_hbm.at[pl.ds(base, bpw)])

    return gather_k(idx, u_grid, v_grid)


def kernel(y, grid_points, u_grid, v_grid, sigma):
    k = grid_points.shape[0]
    kq = _LANES * _UNROLL
    kpad = (k + kq - 1) // kq * kq
    pad = kpad - k
    gxp = jnp.pad(grid_points[:, 0], (0, pad), constant_values=jnp.inf)
    gyp = jnp.pad(grid_points[:, 1], (0, pad), constant_values=jnp.inf)
    gxp = gxp.reshape(1, kpad)
    gyp = gyp.reshape(1, kpad)
    sig = jnp.reshape(sigma, (1, 1)).astype(jnp.float32)

    idx2, g_out = _nn_argmin(y, gxp, gyp, sig)
    idx = idx2.reshape(y.shape[0])
    u_out, v_out = _sc_gather(idx, u_grid, v_grid)
    f_out = jnp.stack([u_out, v_out], axis=1)
    return f_out, g_out


# speedup vs baseline: 1.1410x; 1.1410x over previous
import functools

import jax
import jax.numpy as jnp
from jax import lax
from jax.experimental import pallas as pl
from jax.experimental.pallas import tpu as pltpu
from jax.experimental.pallas import tpu_sc as plsc

_LANES = 128
_FOLD = 16
_NC = 2 * _FOLD
_BIG = 3.0e38


def _nn_body(nblocks, qb, y_ref, gx_ref, gy_ref, sig_ref,
             m_ref, kidx_ref, g_ref):
    yx = y_ref[:, 0:1]
    yy = y_ref[:, 1:2]
    lane = lax.broadcasted_iota(jnp.int32, (1, _LANES), 1).astype(jnp.float32)

    def block(b, carry):
        bv, bb = carry
        mb = None
        for u in range(_FOLD):
            c = b * _FOLD + u
            off = pl.multiple_of(c * _LANES, _LANES)
            gx = gx_ref[:, pl.ds(off, _LANES)]
            gy = gy_ref[:, pl.ds(off, _LANES)]
            dx = yx - gx
            dy = yy - gy
            d = dx * dx + dy * dy
            mb = d if mb is None else jnp.minimum(mb, d)
        upd = mb < bv
        bf = lax.convert_element_type(b, jnp.float32)
        return jnp.where(upd, mb, bv), jnp.where(upd, bf, bb)

    bv0 = jnp.full((qb, _LANES), jnp.inf, jnp.float32)
    bb0 = jnp.zeros((qb, _LANES), jnp.float32)
    bv, bb = lax.fori_loop(0, nblocks, block, (bv0, bb0))

    m = jnp.min(bv, axis=1, keepdims=True)
    flag = bv == m
    l1 = jnp.min(jnp.where(flag, lane, _BIG), axis=1, keepdims=True)
    l2 = jnp.min(jnp.where(flag & (lane > l1), lane, _BIG), axis=1,
                 keepdims=True)
    b1 = jnp.min(jnp.where(lane == l1, bb, _BIG), axis=1, keepdims=True)
    b2 = jnp.min(jnp.where(lane == l2, bb, _BIG), axis=1, keepdims=True)
    blkf = jnp.float32(_LANES * _FOLD)
    kb1 = b1 * blkf + l1
    has2 = l2 < jnp.float32(_LANES)
    kb2 = jnp.where(has2, b2 * blkf + l2, kb1)
    lane32 = lax.broadcasted_iota(jnp.int32, (1, _NC), 1)
    j32 = jnp.where(lane32 < _FOLD, lane32, lane32 - _FOLD)
    j32f = j32.astype(jnp.float32) * jnp.float32(_LANES)
    kbase32 = jnp.where(lane32 < _FOLD, kb1, kb2)
    kidx_ref[...] = kbase32 + j32f
    m_ref[...] = m
    g_ref[...] = jnp.broadcast_to(jnp.exp(sig_ref[0, 0]), g_ref.shape)


def _nn_scan(y, gxp, gyp, sig):
    q = y.shape[0]
    kpad = gxp.shape[1]
    qb = 1024
    nblocks = kpad // (_LANES * _FOLD)
    body = functools.partial(_nn_body, nblocks, qb)
    return pl.pallas_call(
        body,
        grid=(q // qb,),
        in_specs=[
            pl.BlockSpec((qb, 2), lambda i: (i, 0)),
            pl.BlockSpec((1, kpad), lambda i: (0, 0)),
            pl.BlockSpec((1, kpad), lambda i: (0, 0)),
            pl.BlockSpec((1, 1), lambda i: (0, 0)),
        ],
        out_specs=[
            pl.BlockSpec((qb, 1), lambda i: (i, 0)),
            pl.BlockSpec((qb, _NC), lambda i: (i, 0)),
            pl.BlockSpec((qb, 2), lambda i: (i, 0)),
        ],
        out_shape=[
            jax.ShapeDtypeStruct((q, 1), jnp.float32),
            jax.ShapeDtypeStruct((q, _NC), jnp.float32),
            jax.ShapeDtypeStruct((q, 2), jnp.float32),
        ],
    )(y, gxp, gyp, sig)


_CHUNK = 512


def _sc_gather_cand(kidx_flat, gxp, gyp):
    n = kidx_flat.shape[0]
    info = plsc.get_sparse_core_info()
    nw = info.num_cores * info.num_subcores
    bpw = n // nw
    nch = bpw // _CHUNK
    mesh = plsc.VectorSubcoreMesh(core_axis_name="c", subcore_axis_name="s")

    @functools.partial(
        pl.kernel,
        mesh=mesh,
        out_type=[
            jax.ShapeDtypeStruct((n,), jnp.float32),
            jax.ShapeDtypeStruct((n,), jnp.float32),
        ],
        scratch_types=[
            pltpu.VMEM((bpw,), jnp.int32),
            pltpu.VMEM((bpw,), jnp.float32),
            pltpu.VMEM((bpw,), jnp.float32),
            pltpu.SemaphoreType.DMA,
            pltpu.SemaphoreType.DMA,
        ],
    )
    def body(kidx_hbm, gx_hbm, gy_hbm, ox_hbm, oy_hbm, idx_v, xg, yg, s1, s2):
        wid = lax.axis_index("s") * info.num_cores + lax.axis_index("c")
        base = wid * bpw
        pltpu.sync_copy(kidx_hbm.at[pl.ds(base, bpw)], idx_v)

        def chunk(c, carry):
            off = c * _CHUNK
            ca = pltpu.async_copy(gx_hbm.at[idx_v.at[pl.ds(off, _CHUNK)]],
                                  xg.at[pl.ds(off, _CHUNK)], s1)
            cb = pltpu.async_copy(gy_hbm.at[idx_v.at[pl.ds(off, _CHUNK)]],
                                  yg.at[pl.ds(off, _CHUNK)], s2)
            ca.wait()
            cb.wait()
            return carry

        lax.fori_loop(0, nch, chunk, 0)
        pltpu.sync_copy(xg, ox_hbm.at[pl.ds(base, bpw)])
        pltpu.sync_copy(yg, oy_hbm.at[pl.ds(base, bpw)])

    return body(kidx_flat, gxp, gyp)


def _resolve_body(kreal, y_ref, m_ref, kidxf_ref, gxg_ref, gyg_ref, k_ref):
    yx = y_ref[:, 0:1]
    yy = y_ref[:, 1:2]
    dx = yx - gxg_ref[...]
    dy = yy - gyg_ref[...]
    d = dx * dx + dy * dy
    hit = d == m_ref[...]
    ksel = jnp.min(jnp.where(hit, kidxf_ref[...], _BIG), axis=1,
                   keepdims=True)
    ksel = jnp.minimum(ksel, jnp.float32(kreal - 1))
    k_ref[...] = ksel.astype(jnp.int32)


def _nn_resolve(y, m, kidxf, gxg, gyg, kreal):
    q = y.shape[0]
    qb = 1024
    body = functools.partial(_resolve_body, kreal)
    return pl.pallas_call(
        body,
        grid=(q // qb,),
        in_specs=[
            pl.BlockSpec((qb, 2), lambda i: (i, 0)),
            pl.BlockSpec((qb, 1), lambda i: (i, 0)),
            pl.BlockSpec((qb, _NC), lambda i: (i, 0)),
            pl.BlockSpec((qb, _NC), lambda i: (i, 0)),
            pl.BlockSpec((qb, _NC), lambda i: (i, 0)),
        ],
        out_specs=[pl.BlockSpec((qb, 1), lambda i: (i, 0))],
        out_shape=[jax.ShapeDtypeStruct((q, 1), jnp.int32)],
    )(y, m, kidxf, gxg, gyg)[0]


def _sc_gather_uv(idx, u_grid, v_grid):
    q = idx.shape[0]
    info = plsc.get_sparse_core_info()
    nw = info.num_cores * info.num_subcores
    bpw = q // nw
    mesh = plsc.VectorSubcoreMesh(core_axis_name="c", subcore_axis_name="s")

    @functools.partial(
        pl.kernel,
        mesh=mesh,
        out_type=[
            jax.ShapeDtypeStruct((q,), jnp.float32),
            jax.ShapeDtypeStruct((q,), jnp.float32),
        ],
        scratch_types=[
            pltpu.VMEM((bpw,), jnp.int32),
            pltpu.VMEM((bpw,), jnp.float32),
            pltpu.VMEM((bpw,), jnp.float32),
            pltpu.SemaphoreType.DMA,
            pltpu.SemaphoreType.DMA,
        ],
    )
    def body(idx_hbm, u_hbm, v_hbm, uo_hbm, vo_hbm, idx_v, u_v, v_v, s1, s2):
        wid = lax.axis_index("s") * info.num_cores + lax.axis_index("c")
        base = wid * bpw
        pltpu.sync_copy(idx_hbm.at[pl.ds(base, bpw)], idx_v)
        cu = pltpu.async_copy(u_hbm.at[idx_v], u_v, s1)
        cv = pltpu.async_copy(v_hbm.at[idx_v], v_v, s2)
        cu.wait()
        cv.wait()
        pltpu.sync_copy(u_v, uo_hbm.at[pl.ds(base, bpw)])
        pltpu.sync_copy(v_v, vo_hbm.at[pl.ds(base, bpw)])

    return body(idx, u_grid, v_grid)


def kernel(y, grid_points, u_grid, v_grid, sigma):
    k = grid_points.shape[0]
    kq = _LANES * _FOLD
    kpad = (k + kq - 1) // kq * kq
    pad = kpad - k
    gxp = jnp.pad(grid_points[:, 0], (0, pad), constant_values=jnp.inf)
    gyp = jnp.pad(grid_points[:, 1], (0, pad), constant_values=jnp.inf)
    sig = jnp.reshape(sigma, (1, 1)).astype(jnp.float32)

    m, kidxf, g_out = _nn_scan(y, gxp.reshape(1, kpad), gyp.reshape(1, kpad),
                               sig)
    qn = y.shape[0]
    kidx_flat = kidxf.astype(jnp.int32).reshape(qn * _NC)
    gxg, gyg = _sc_gather_cand(kidx_flat, gxp, gyp)
    kfin = _nn_resolve(y, m, kidxf, gxg.reshape(qn, _NC),
                       gyg.reshape(qn, _NC), k)
    u_out, v_out = _sc_gather_uv(kfin.reshape(qn), u_grid, v_grid)
    f_out = jnp.stack([u_out, v_out], axis=1)
    return f_out, g_out


# speedup vs baseline: 1.1471x; 1.0054x over previous
import functools

import jax
import jax.numpy as jnp
from jax import lax
from jax.experimental import pallas as pl
from jax.experimental.pallas import tpu as pltpu
from jax.experimental.pallas import tpu_sc as plsc

_LANES = 128
_FOLD = 16
_NC = 2 * _FOLD
_BIG = 3.0e38


def _nn_body(nblocks, qb, y_ref, gx_ref, gy_ref, sig_ref,
             m_ref, kidx_ref, g_ref):
    yx = y_ref[:, 0:1]
    yy = y_ref[:, 1:2]
    lane = lax.broadcasted_iota(jnp.int32, (1, _LANES), 1).astype(jnp.float32)

    def block(b, carry):
        bv, bb = carry
        mb = None
        for u in range(_FOLD):
            c = b * _FOLD + u
            off = pl.multiple_of(c * _LANES, _LANES)
            gx = gx_ref[:, pl.ds(off, _LANES)]
            gy = gy_ref[:, pl.ds(off, _LANES)]
            dx = yx - gx
            dy = yy - gy
            d = dx * dx + dy * dy
            mb = d if mb is None else jnp.minimum(mb, d)
        upd = mb < bv
        bf = lax.convert_element_type(b, jnp.float32)
        return jnp.where(upd, mb, bv), jnp.where(upd, bf, bb)

    bv0 = jnp.full((qb, _LANES), jnp.inf, jnp.float32)
    bb0 = jnp.zeros((qb, _LANES), jnp.float32)
    bv, bb = lax.fori_loop(0, nblocks, block, (bv0, bb0))

    m = jnp.min(bv, axis=1, keepdims=True)
    flag = bv == m
    l1 = jnp.min(jnp.where(flag, lane, _BIG), axis=1, keepdims=True)
    l2 = jnp.min(jnp.where(flag & (lane > l1), lane, _BIG), axis=1,
                 keepdims=True)
    b1 = jnp.min(jnp.where(lane == l1, bb, _BIG), axis=1, keepdims=True)
    b2 = jnp.min(jnp.where(lane == l2, bb, _BIG), axis=1, keepdims=True)
    blkf = jnp.float32(_LANES * _FOLD)
    kb1 = b1 * blkf + l1
    has2 = l2 < jnp.float32(_LANES)
    kb2 = jnp.where(has2, b2 * blkf + l2, kb1)
    lane32 = lax.broadcasted_iota(jnp.int32, (1, _NC), 1)
    j32 = jnp.where(lane32 < _FOLD, lane32, lane32 - _FOLD)
    j32f = j32.astype(jnp.float32) * jnp.float32(_LANES)
    kbase32 = jnp.where(lane32 < _FOLD, kb1, kb2)
    kidx_ref[...] = kbase32 + j32f
    m_ref[...] = m
    g_ref[...] = jnp.broadcast_to(jnp.exp(sig_ref[0, 0]), g_ref.shape)


def _nn_scan(y, gxp, gyp, sig):
    q = y.shape[0]
    kpad = gxp.shape[1]
    qb = 2048
    nblocks = kpad // (_LANES * _FOLD)
    body = functools.partial(_nn_body, nblocks, qb)
    return pl.pallas_call(
        body,
        grid=(q // qb,),
        in_specs=[
            pl.BlockSpec((qb, 2), lambda i: (i, 0)),
            pl.BlockSpec((1, kpad), lambda i: (0, 0)),
            pl.BlockSpec((1, kpad), lambda i: (0, 0)),
            pl.BlockSpec((1, 1), lambda i: (0, 0)),
        ],
        out_specs=[
            pl.BlockSpec((qb, 1), lambda i: (i, 0)),
            pl.BlockSpec((qb, _NC), lambda i: (i, 0)),
            pl.BlockSpec((qb, 2), lambda i: (i, 0)),
        ],
        out_shape=[
            jax.ShapeDtypeStruct((q, 1), jnp.float32),
            jax.ShapeDtypeStruct((q, _NC), jnp.float32),
            jax.ShapeDtypeStruct((q, 2), jnp.float32),
        ],
    )(y, gxp, gyp, sig)


_CHUNK = 512


def _sc_gather_cand(kidx_flat, gxp, gyp):
    n = kidx_flat.shape[0]
    info = plsc.get_sparse_core_info()
    nw = info.num_cores * info.num_subcores
    bpw = n // nw
    nch = bpw // _CHUNK
    mesh = plsc.VectorSubcoreMesh(core_axis_name="c", subcore_axis_name="s")

    @functools.partial(
        pl.kernel,
        mesh=mesh,
        out_type=[
            jax.ShapeDtypeStruct((n,), jnp.float32),
            jax.ShapeDtypeStruct((n,), jnp.float32),
        ],
        scratch_types=[
            pltpu.VMEM((bpw,), jnp.int32),
            pltpu.VMEM((bpw,), jnp.float32),
            pltpu.VMEM((bpw,), jnp.float32),
            pltpu.SemaphoreType.DMA,
            pltpu.SemaphoreType.DMA,
        ],
    )
    def body(kidx_hbm, gx_hbm, gy_hbm, ox_hbm, oy_hbm, idx_v, xg, yg, s1, s2):
        wid = lax.axis_index("s") * info.num_cores + lax.axis_index("c")
        base = wid * bpw
        pltpu.sync_copy(kidx_hbm.at[pl.ds(base, bpw)], idx_v)

        def chunk(c, carry):
            off = c * _CHUNK
            ca = pltpu.async_copy(gx_hbm.at[idx_v.at[pl.ds(off, _CHUNK)]],
                                  xg.at[pl.ds(off, _CHUNK)], s1)
            cb = pltpu.async_copy(gy_hbm.at[idx_v.at[pl.ds(off, _CHUNK)]],
                                  yg.at[pl.ds(off, _CHUNK)], s2)
            ca.wait()
            cb.wait()
            return carry

        lax.fori_loop(0, nch, chunk, 0)
        pltpu.sync_copy(xg, ox_hbm.at[pl.ds(base, bpw)])
        pltpu.sync_copy(yg, oy_hbm.at[pl.ds(base, bpw)])

    return body(kidx_flat, gxp, gyp)


def _resolve_body(kreal, y_ref, m_ref, kidxf_ref, gxg_ref, gyg_ref, k_ref):
    yx = y_ref[:, 0:1]
    yy = y_ref[:, 1:2]
    dx = yx - gxg_ref[...]
    dy = yy - gyg_ref[...]
    d = dx * dx + dy * dy
    hit = d == m_ref[...]
    ksel = jnp.min(jnp.where(hit, kidxf_ref[...], _BIG), axis=1,
                   keepdims=True)
    ksel = jnp.minimum(ksel, jnp.float32(kreal - 1))
    k_ref[...] = ksel.astype(jnp.int32)


def _nn_resolve(y, m, kidxf, gxg, gyg, kreal):
    q = y.shape[0]
    qb = 1024
    body = functools.partial(_resolve_body, kreal)
    return pl.pallas_call(
        body,
        grid=(q // qb,),
        in_specs=[
            pl.BlockSpec((qb, 2), lambda i: (i, 0)),
            pl.BlockSpec((qb, 1), lambda i: (i, 0)),
            pl.BlockSpec((qb, _NC), lambda i: (i, 0)),
            pl.BlockSpec((qb, _NC), lambda i: (i, 0)),
            pl.BlockSpec((qb, _NC), lambda i: (i, 0)),
        ],
        out_specs=[pl.BlockSpec((qb, 1), lambda i: (i, 0))],
        out_shape=[jax.ShapeDtypeStruct((q, 1), jnp.int32)],
    )(y, m, kidxf, gxg, gyg)[0]


def _sc_gather_uv(idx, u_grid, v_grid):
    q = idx.shape[0]
    info = plsc.get_sparse_core_info()
    nw = info.num_cores * info.num_subcores
    bpw = q // nw
    mesh = plsc.VectorSubcoreMesh(core_axis_name="c", subcore_axis_name="s")

    @functools.partial(
        pl.kernel,
        mesh=mesh,
        out_type=[
            jax.ShapeDtypeStruct((q,), jnp.float32),
            jax.ShapeDtypeStruct((q,), jnp.float32),
        ],
        scratch_types=[
            pltpu.VMEM((bpw,), jnp.int32),
            pltpu.VMEM((bpw,), jnp.float32),
            pltpu.VMEM((bpw,), jnp.float32),
            pltpu.SemaphoreType.DMA,
            pltpu.SemaphoreType.DMA,
        ],
    )
    def body(idx_hbm, u_hbm, v_hbm, uo_hbm, vo_hbm, idx_v, u_v, v_v, s1, s2):
        wid = lax.axis_index("s") * info.num_cores + lax.axis_index("c")
        base = wid * bpw
        pltpu.sync_copy(idx_hbm.at[pl.ds(base, bpw)], idx_v)
        cu = pltpu.async_copy(u_hbm.at[idx_v], u_v, s1)
        cv = pltpu.async_copy(v_hbm.at[idx_v], v_v, s2)
        cu.wait()
        cv.wait()
        pltpu.sync_copy(u_v, uo_hbm.at[pl.ds(base, bpw)])
        pltpu.sync_copy(v_v, vo_hbm.at[pl.ds(base, bpw)])

    return body(idx, u_grid, v_grid)


def kernel(y, grid_points, u_grid, v_grid, sigma):
    k = grid_points.shape[0]
    kq = _LANES * _FOLD
    kpad = (k + kq - 1) // kq * kq
    pad = kpad - k
    gxp = jnp.pad(grid_points[:, 0], (0, pad), constant_values=jnp.inf)
    gyp = jnp.pad(grid_points[:, 1], (0, pad), constant_values=jnp.inf)
    sig = jnp.reshape(sigma, (1, 1)).astype(jnp.float32)

    m, kidxf, g_out = _nn_scan(y, gxp.reshape(1, kpad), gyp.reshape(1, kpad),
                               sig)
    qn = y.shape[0]
    kidx_flat = kidxf.astype(jnp.int32).reshape(qn * _NC)
    gxg, gyg = _sc_gather_cand(kidx_flat, gxp, gyp)
    kfin = _nn_resolve(y, m, kidxf, gxg.reshape(qn, _NC),
                       gyg.reshape(qn, _NC), k)
    u_out, v_out = _sc_gather_uv(kfin.reshape(qn), u_grid, v_grid)
    f_out = jnp.stack([u_out, v_out], axis=1)
    return f_out, g_out


# speedup vs baseline: 1.1745x; 1.0239x over previous
import functools

import jax
import jax.numpy as jnp
from jax import lax
from jax.experimental import pallas as pl
from jax.experimental.pallas import tpu as pltpu
from jax.experimental.pallas import tpu_sc as plsc

_LANES = 128
_FOLD = 16
_NC = 2 * _FOLD
_BIG = 3.0e38


def _nn_body(nblocks, qb, y_ref, gx_ref, gy_ref, sig_ref,
             m_ref, kidx_ref, g_ref):
    yx = y_ref[:, 0:1]
    yy = y_ref[:, 1:2]
    lane = lax.broadcasted_iota(jnp.int32, (1, _LANES), 1).astype(jnp.float32)

    def one_block(b, bv, bb):
        mb = None
        for u in range(_FOLD):
            c = b * _FOLD + u
            off = pl.multiple_of(c * _LANES, _LANES)
            gx = gx_ref[:, pl.ds(off, _LANES)]
            gy = gy_ref[:, pl.ds(off, _LANES)]
            dx = yx - gx
            dy = yy - gy
            d = dx * dx + dy * dy
            mb = d if mb is None else jnp.minimum(mb, d)
        upd = mb < bv
        bf = lax.convert_element_type(b, jnp.float32)
        return jnp.where(upd, mb, bv), jnp.where(upd, bf, bb)

    def block2(s2, carry):
        bv, bb = carry
        bv, bb = one_block(s2 * 2, bv, bb)
        return one_block(s2 * 2 + 1, bv, bb)

    bv0 = jnp.full((qb, _LANES), jnp.inf, jnp.float32)
    bb0 = jnp.zeros((qb, _LANES), jnp.float32)
    bv, bb = lax.fori_loop(0, nblocks // 2, block2, (bv0, bb0))

    m = jnp.min(bv, axis=1, keepdims=True)
    flag = bv == m
    l1 = jnp.min(jnp.where(flag, lane, _BIG), axis=1, keepdims=True)
    l2 = jnp.min(jnp.where(flag & (lane > l1), lane, _BIG), axis=1,
                 keepdims=True)
    b1 = jnp.min(jnp.where(lane == l1, bb, _BIG), axis=1, keepdims=True)
    b2 = jnp.min(jnp.where(lane == l2, bb, _BIG), axis=1, keepdims=True)
    blkf = jnp.float32(_LANES * _FOLD)
    kb1 = b1 * blkf + l1
    has2 = l2 < jnp.float32(_LANES)
    kb2 = jnp.where(has2, b2 * blkf + l2, kb1)
    lane32 = lax.broadcasted_iota(jnp.int32, (1, _NC), 1)
    j32 = jnp.where(lane32 < _FOLD, lane32, lane32 - _FOLD)
    j32f = j32.astype(jnp.float32) * jnp.float32(_LANES)
    kbase32 = jnp.where(lane32 < _FOLD, kb1, kb2)
    kidx_ref[...] = kbase32 + j32f
    m_ref[...] = m
    g_ref[...] = jnp.broadcast_to(jnp.exp(sig_ref[0, 0]), g_ref.shape)


def _nn_scan(y, gxp, gyp, sig):
    q = y.shape[0]
    kpad = gxp.shape[1]
    qb = 2048
    nblocks = kpad // (_LANES * _FOLD)
    body = functools.partial(_nn_body, nblocks, qb)
    return pl.pallas_call(
        body,
        grid=(q // qb,),
        in_specs=[
            pl.BlockSpec((qb, 2), lambda i: (i, 0)),
            pl.BlockSpec((1, kpad), lambda i: (0, 0)),
            pl.BlockSpec((1, kpad), lambda i: (0, 0)),
            pl.BlockSpec((1, 1), lambda i: (0, 0)),
        ],
        out_specs=[
            pl.BlockSpec((qb, 1), lambda i: (i, 0)),
            pl.BlockSpec((qb, _NC), lambda i: (i, 0)),
            pl.BlockSpec((qb, 2), lambda i: (i, 0)),
        ],
        out_shape=[
            jax.ShapeDtypeStruct((q, 1), jnp.float32),
            jax.ShapeDtypeStruct((q, _NC), jnp.float32),
            jax.ShapeDtypeStruct((q, 2), jnp.float32),
        ],
    )(y, gxp, gyp, sig)


_CHUNK = 512


def _sc_gather_cand(kidx_flat, gxp, gyp):
    n = kidx_flat.shape[0]
    info = plsc.get_sparse_core_info()
    nw = info.num_cores * info.num_subcores
    bpw = n // nw
    nch = bpw // _CHUNK
    mesh = plsc.VectorSubcoreMesh(core_axis_name="c", subcore_axis_name="s")

    @functools.partial(
        pl.kernel,
        mesh=mesh,
        out_type=[
            jax.ShapeDtypeStruct((n,), jnp.float32),
            jax.ShapeDtypeStruct((n,), jnp.float32),
        ],
        scratch_types=[
            pltpu.VMEM((bpw,), jnp.int32),
            pltpu.VMEM((bpw,), jnp.float32),
            pltpu.VMEM((bpw,), jnp.float32),
            pltpu.SemaphoreType.DMA,
            pltpu.SemaphoreType.DMA,
        ],
    )
    def body(kidx_hbm, gx_hbm, gy_hbm, ox_hbm, oy_hbm, idx_v, xg, yg, s1, s2):
        wid = lax.axis_index("s") * info.num_cores + lax.axis_index("c")
        base = wid * bpw
        pltpu.sync_copy(kidx_hbm.at[pl.ds(base, bpw)], idx_v)

        def chunk(c, carry):
            off = c * _CHUNK
            ca = pltpu.async_copy(gx_hbm.at[idx_v.at[pl.ds(off, _CHUNK)]],
                                  xg.at[pl.ds(off, _CHUNK)], s1)
            cb = pltpu.async_copy(gy_hbm.at[idx_v.at[pl.ds(off, _CHUNK)]],
                                  yg.at[pl.ds(off, _CHUNK)], s2)
            ca.wait()
            cb.wait()
            return carry

        lax.fori_loop(0, nch, chunk, 0)
        pltpu.sync_copy(xg, ox_hbm.at[pl.ds(base, bpw)])
        pltpu.sync_copy(yg, oy_hbm.at[pl.ds(base, bpw)])

    return body(kidx_flat, gxp, gyp)


def _resolve_body(kreal, y_ref, m_ref, kidxf_ref, gxg_ref, gyg_ref, k_ref):
    yx = y_ref[:, 0:1]
    yy = y_ref[:, 1:2]
    dx = yx - gxg_ref[...]
    dy = yy - gyg_ref[...]
    d = dx * dx + dy * dy
    hit = d == m_ref[...]
    ksel = jnp.min(jnp.where(hit, kidxf_ref[...], _BIG), axis=1,
                   keepdims=True)
    ksel = jnp.minimum(ksel, jnp.float32(kreal - 1))
    k_ref[...] = ksel.astype(jnp.int32)


def _nn_resolve(y, m, kidxf, gxg, gyg, kreal):
    q = y.shape[0]
    qb = 1024
    body = functools.partial(_resolve_body, kreal)
    return pl.pallas_call(
        body,
        grid=(q // qb,),
        in_specs=[
            pl.BlockSpec((qb, 2), lambda i: (i, 0)),
            pl.BlockSpec((qb, 1), lambda i: (i, 0)),
            pl.BlockSpec((qb, _NC), lambda i: (i, 0)),
            pl.BlockSpec((qb, _NC), lambda i: (i, 0)),
            pl.BlockSpec((qb, _NC), lambda i: (i, 0)),
        ],
        out_specs=[pl.BlockSpec((qb, 1), lambda i: (i, 0))],
        out_shape=[jax.ShapeDtypeStruct((q, 1), jnp.int32)],
    )(y, m, kidxf, gxg, gyg)[0]


def _sc_gather_uv(idx, u_grid, v_grid):
    q = idx.shape[0]
    info = plsc.get_sparse_core_info()
    nw = info.num_cores * info.num_subcores
    bpw = q // nw
    mesh = plsc.VectorSubcoreMesh(core_axis_name="c", subcore_axis_name="s")

    @functools.partial(
        pl.kernel,
        mesh=mesh,
        out_type=[
            jax.ShapeDtypeStruct((q,), jnp.float32),
            jax.ShapeDtypeStruct((q,), jnp.float32),
        ],
        scratch_types=[
            pltpu.VMEM((bpw,), jnp.int32),
            pltpu.VMEM((bpw,), jnp.float32),
            pltpu.VMEM((bpw,), jnp.float32),
            pltpu.SemaphoreType.DMA,
            pltpu.SemaphoreType.DMA,
        ],
    )
    def body(idx_hbm, u_hbm, v_hbm, uo_hbm, vo_hbm, idx_v, u_v, v_v, s1, s2):
        wid = lax.axis_index("s") * info.num_cores + lax.axis_index("c")
        base = wid * bpw
        pltpu.sync_copy(idx_hbm.at[pl.ds(base, bpw)], idx_v)
        cu = pltpu.async_copy(u_hbm.at[idx_v], u_v, s1)
        cv = pltpu.async_copy(v_hbm.at[idx_v], v_v, s2)
        cu.wait()
        cv.wait()
        pltpu.sync_copy(u_v, uo_hbm.at[pl.ds(base, bpw)])
        pltpu.sync_copy(v_v, vo_hbm.at[pl.ds(base, bpw)])

    return body(idx, u_grid, v_grid)


def kernel(y, grid_points, u_grid, v_grid, sigma):
    k = grid_points.shape[0]
    kq = _LANES * _FOLD * 2
    kpad = (k + kq - 1) // kq * kq
    pad = kpad - k
    gxp = jnp.pad(grid_points[:, 0], (0, pad), constant_values=jnp.inf)
    gyp = jnp.pad(grid_points[:, 1], (0, pad), constant_values=jnp.inf)
    sig = jnp.reshape(sigma, (1, 1)).astype(jnp.float32)

    m, kidxf, g_out = _nn_scan(y, gxp.reshape(1, kpad), gyp.reshape(1, kpad),
                               sig)
    qn = y.shape[0]
    kidx_flat = kidxf.astype(jnp.int32).reshape(qn * _NC)
    gxg, gyg = _sc_gather_cand(kidx_flat, gxp, gyp)
    kfin = _nn_resolve(y, m, kidxf, gxg.reshape(qn, _NC),
                       gyg.reshape(qn, _NC), k)
    u_out, v_out = _sc_gather_uv(kfin.reshape(qn), u_grid, v_grid)
    f_out = jnp.stack([u_out, v_out], axis=1)
    return f_out, g_out
